# Initial kernel scaffold; baseline (speedup 1.0000x reference)
import jax
import jax.numpy as jnp
from jax.experimental import pallas as pl


def kernel(static_dense_x, static_sparse_x, dynamic_dense_x, dynamic_sparse_x, edges, weights, static_emb_0, static_emb_1, dyn_emb_0, dyn_emb_1, W1, b1, W2, b2):
    raise NotImplementedError("write your pallas kernel here")


# speedup vs baseline: 3.9608x; 3.9608x over previous
import functools

import jax
import jax.numpy as jnp
from jax import lax
from jax.experimental import pallas as pl
from jax.experimental.pallas import tpu as pltpu
from jax.experimental.pallas import tpu_sc as plsc

N = 10000
E = 320000
VOCAB = 1000
ENT = 16

NC = 2
NS = 16
NW = NC * NS
L = 16

EW = E // NW
CK = 80
NCHUNK = EW // CK

OWN = 640

NB = 80
NBLK = N // NB


def _mesh():
    return plsc.VectorSubcoreMesh(core_axis_name="c", subcore_axis_name="s",
                                  num_cores=NC, num_subcores=NS)


def _emb_body(i0, i1, i2, i3, t0, t1, t2, t3, o0, o1, o2, o3,
              ibuf, gbuf, sem):
    c = lax.axis_index("c")
    s = lax.axis_index("s")
    wid = c * NS + s
    idxs = (i0, i1, i2, i3)
    tabs = (t0, t1, t2, t3)
    outs = (o0, o1, o2, o3)
    for j in range(4):
        blk = wid + j * NW

        @pl.when(blk < NBLK)
        def _():
            r0 = blk * NB
            for t in range(4):
                pltpu.sync_copy(idxs[t].at[pl.ds(r0, NB)], ibuf)
                pltpu.async_copy(tabs[t].at[ibuf], gbuf, sem).wait()
                pltpu.sync_copy(gbuf, outs[t].at[pl.ds(r0, NB)])


def _emb_gather(ds0, ds1, ss0, ss1, dt0, dt1, st0, st1):
    f = pl.kernel(
        _emb_body,
        out_type=tuple(jax.ShapeDtypeStruct((N, ENT), jnp.float32)
                       for _ in range(4)),
        mesh=_mesh(),
        compiler_params=pltpu.CompilerParams(use_tc_tiling_on_sc=False),
        scratch_types=[
            pltpu.VMEM((NB,), jnp.int32),
            pltpu.VMEM((NB, ENT), jnp.float32),
            pltpu.SemaphoreType.DMA,
        ],
    )
    return f(ds0, ds1, ss0, ss1, dt0, dt1, st0, st1)


def _prop_body(D, x, src, dst, wgt, out, acc, sidx, didx, wbuf,
               rows, sem):
    c = lax.axis_index("c")
    s = lax.axis_index("s")
    wid = c * NS + s
    NV = D // L

    def _z(i, _):
        for v in range(NV):
            rows[i, pl.ds(v * L, L)] = jnp.zeros((L,), jnp.float32)
        return _

    lax.fori_loop(0, CK, _z, None)
    ncp = jnp.where(s < NS - 1, OWN // 80, (N - (NS - 1) * OWN) // 80)

    def _zc(i, _):
        pltpu.sync_copy(rows, acc.at[pl.ds(s * OWN + i * 80, 80)])
        return _

    lax.fori_loop(0, ncp, _zc, None)
    plsc.subcore_barrier()

    e_base = wid * EW

    def _chunk(k, _):
        e0 = e_base + k * CK
        pltpu.sync_copy(src.at[pl.ds(e0, CK)], sidx)
        pltpu.sync_copy(dst.at[pl.ds(e0, CK)], didx.at[0])
        pltpu.sync_copy(wgt.at[pl.ds(e0, CK)], wbuf)
        pltpu.async_copy(x.at[sidx], rows, sem).wait()

        def _scale(e, _):
            wspl = plsc.load_gather(wbuf, [jnp.full((L,), e, jnp.int32)])
            for v in range(NV):
                sl = pl.ds(v * L, L)
                rows[e, sl] = rows[e, sl] * wspl
            return _

        lax.fori_loop(0, CK, _scale, None)
        pltpu.sync_copy(rows, acc.at[didx.at[0]], add=True)
        return _

    lax.fori_loop(0, NCHUNK, _chunk, None)
    plsc.subcore_barrier()

    def _wc(i, _):
        pltpu.sync_copy(acc.at[pl.ds(s * OWN + i * 80, 80)],
                        out.at[pl.ds(c * N + s * OWN + i * 80, 80)])
        return _

    lax.fori_loop(0, ncp, _wc, None)


def _prop(x, src, dst, wgt, D):
    f = pl.kernel(
        functools.partial(_prop_body, D),
        out_type=jax.ShapeDtypeStruct((2 * N, D), jnp.float32),
        mesh=_mesh(),
        compiler_params=pltpu.CompilerParams(use_tc_tiling_on_sc=False,
                                             needs_layout_passes=False),
        scratch_types=[
            pltpu.VMEM_SHARED((N, D), jnp.float32),
            pltpu.VMEM((CK,), jnp.int32),
            pltpu.VMEM((1, CK), jnp.int32),
            pltpu.VMEM((CK,), jnp.float32),
            pltpu.VMEM((CK, D), jnp.float32),
            pltpu.SemaphoreType.DMA,
        ],
    )
    return f(x, src, dst, wgt)


RB = 1000


def _mix_body(g0, g1, dd, g2, g3, sd, w1, o):
    x = jnp.dot(g0[...], w1[0:16, :], preferred_element_type=jnp.float32)
    x += jnp.dot(g1[...], w1[16:32, :], preferred_element_type=jnp.float32)
    x += jnp.dot(dd[...], w1[32:64, :], preferred_element_type=jnp.float32)
    x += jnp.dot(g2[...], w1[64:80, :], preferred_element_type=jnp.float32)
    x += jnp.dot(g3[...], w1[80:96, :], preferred_element_type=jnp.float32)
    x += jnp.dot(sd[...], w1[96:128, :], preferred_element_type=jnp.float32)
    o[...] = x


def _mix(g0, g1, dd, g2, g3, sd, w1):
    grid = (N // RB,)
    bs_e = pl.BlockSpec((RB, ENT), lambda i: (i, 0))
    bs_d = pl.BlockSpec((RB, 32), lambda i: (i, 0))
    bs_w = pl.BlockSpec((128, 128), lambda i: (0, 0))
    return pl.pallas_call(
        _mix_body,
        grid=grid,
        in_specs=[bs_e, bs_e, bs_d, bs_e, bs_e, bs_d, bs_w],
        out_specs=pl.BlockSpec((RB, 128), lambda i: (i, 0)),
        out_shape=jax.ShapeDtypeStruct((N, 128), jnp.float32),
    )(g0, g1, dd, g2, g3, sd, w1)


def _act_body(pa, pb, b1, w2, o):
    h = jnp.maximum(pa[...] + pb[...] + b1[...], 0.0)
    o[...] = jnp.dot(h, w2[...], preferred_element_type=jnp.float32)


def _act(p1, b1, w2):
    grid = (N // RB,)
    return pl.pallas_call(
        _act_body,
        grid=grid,
        in_specs=[
            pl.BlockSpec((RB, 128), lambda i: (i, 0)),
            pl.BlockSpec((RB, 128), lambda i: (i + N // RB, 0)),
            pl.BlockSpec((1, 128), lambda i: (0, 0)),
            pl.BlockSpec((128, 64), lambda i: (0, 0)),
        ],
        out_specs=pl.BlockSpec((RB, 64), lambda i: (i, 0)),
        out_shape=jax.ShapeDtypeStruct((N, 64), jnp.float32),
    )(p1, p1, b1.reshape(1, 128), w2)


def _fin_body(pa, pb, b2, o):
    o[...] = pa[...] + pb[...] + b2[...]


def _fin(p2, b2):
    grid = (N // RB,)
    return pl.pallas_call(
        _fin_body,
        grid=grid,
        in_specs=[
            pl.BlockSpec((RB, 64), lambda i: (i, 0)),
            pl.BlockSpec((RB, 64), lambda i: (i + N // RB, 0)),
            pl.BlockSpec((1, 64), lambda i: (0, 0)),
        ],
        out_specs=pl.BlockSpec((RB, 64), lambda i: (i, 0)),
        out_shape=jax.ShapeDtypeStruct((N, 64), jnp.float32),
    )(p2, p2, b2.reshape(1, 64))


def kernel(static_dense_x, static_sparse_x, dynamic_dense_x, dynamic_sparse_x,
           edges, weights, static_emb_0, static_emb_1, dyn_emb_0, dyn_emb_1,
           W1, b1, W2, b2):
    ss0 = static_sparse_x[:, 0].astype(jnp.int32)
    ss1 = static_sparse_x[:, 1].astype(jnp.int32)
    ds0 = dynamic_sparse_x[0, :, 0].astype(jnp.int32)
    ds1 = dynamic_sparse_x[0, :, 1].astype(jnp.int32)
    src = edges[0, 0].astype(jnp.int32)
    dst = edges[0, 1].astype(jnp.int32)
    w = weights[0]
    ddx = dynamic_dense_x[0]

    g0, g1, g2, g3 = _emb_gather(ds0, ds1, ss0, ss1,
                                 dyn_emb_0, dyn_emb_1,
                                 static_emb_0, static_emb_1)
    x1 = _mix(g0, g1, ddx, g2, g3, static_dense_x, W1)
    p1 = _prop(x1, src, dst, w, 128)
    g = _act(p1, b1, W2)
    p2 = _prop(g, src, dst, w, 64)
    return _fin(p2, b2)


# speedup vs baseline: 6.5924x; 1.6644x over previous
import functools

import jax
import jax.numpy as jnp
from jax import lax
from jax.experimental import pallas as pl
from jax.experimental.pallas import tpu as pltpu
from jax.experimental.pallas import tpu_sc as plsc

N = 10000
E = 320000
VOCAB = 1000
ENT = 16

NC = 2
NS = 16
NW = NC * NS
L = 16

EW = E // NW
CK = 80
NCHUNK = EW // CK
NBUF = 4

OWN = 640

NB = 80
NBLK = N // NB


def _mesh():
    return plsc.VectorSubcoreMesh(core_axis_name="c", subcore_axis_name="s",
                                  num_cores=NC, num_subcores=NS)


def _emb_body(i0, i1, i2, i3, t0, t1, t2, t3, o0, o1, o2, o3,
              ibuf, gbuf, sem):
    c = lax.axis_index("c")
    s = lax.axis_index("s")
    wid = c * NS + s
    idxs = (i0, i1, i2, i3)
    tabs = (t0, t1, t2, t3)
    outs = (o0, o1, o2, o3)
    for j in range(4):
        blk = wid + j * NW

        @pl.when(blk < NBLK)
        def _():
            r0 = blk * NB
            for t in range(4):
                pltpu.sync_copy(idxs[t].at[pl.ds(r0, NB)], ibuf)
                pltpu.async_copy(tabs[t].at[ibuf], gbuf, sem).wait()
                pltpu.sync_copy(gbuf, outs[t].at[pl.ds(r0, NB)])


def _emb_gather(ds0, ds1, ss0, ss1, dt0, dt1, st0, st1):
    f = pl.kernel(
        _emb_body,
        out_type=tuple(jax.ShapeDtypeStruct((N, ENT), jnp.float32)
                       for _ in range(4)),
        mesh=_mesh(),
        compiler_params=pltpu.CompilerParams(use_tc_tiling_on_sc=False),
        scratch_types=[
            pltpu.VMEM((NB,), jnp.int32),
            pltpu.VMEM((NB, ENT), jnp.float32),
            pltpu.SemaphoreType.DMA,
        ],
    )
    return f(ds0, ds1, ss0, ss1, dt0, dt1, st0, st1)


def _prop_body(D, x, edata, out, acc, ebuf,
               r0, r1, r2, r3, g0, g1, g2, g3, s0, s1, s2, s3,
               i0, i1, i2, i3):
    c = lax.axis_index("c")
    s = lax.axis_index("s")
    wid = c * NS + s
    NV = D // L
    rows = (r0, r1, r2, r3)
    gsem = (g0, g1, g2, g3)
    ssem = (s0, s1, s2, s3)
    isem = (i0, i1, i2, i3)

    def _z(i, _):
        for v in range(NV):
            r0[i, pl.ds(v * L, L)] = jnp.zeros((L,), jnp.float32)
        return _

    lax.fori_loop(0, CK, _z, None)
    ncp = jnp.where(s < NS - 1, OWN // 80, (N - (NS - 1) * OWN) // 80)

    def _zc(i, _):
        pltpu.sync_copy(r0, acc.at[pl.ds(s * OWN + i * 80, 80)])
        return _

    lax.fori_loop(0, ncp, _zc, None)

    erow = wid * NCHUNK
    pltpu.sync_copy(edata.at[erow], ebuf.at[0])
    pltpu.async_copy(edata.at[erow + 1], ebuf.at[1], isem[1])
    pltpu.async_copy(x.at[ebuf.at[0, 0]], r0, g0)
    plsc.subcore_barrier()

    def _body(k, b):
        pltpu.make_async_copy(x.at[ebuf.at[b, 0]], rows[b], gsem[b]).wait()

        def _scale(e, _):
            wspl = plsc.bitcast(
                plsc.load_gather(
                    ebuf, [jnp.full((L,), b, jnp.int32),
                           jnp.full((L,), 2, jnp.int32),
                           jnp.full((L,), e, jnp.int32)]), jnp.float32)
            for v in range(NV):
                sl = pl.ds(v * L, L)
                rows[b][e, sl] = rows[b][e, sl] * wspl
            return _

        lax.fori_loop(0, CK, _scale, None, unroll=5)
        pltpu.async_copy(rows[b], acc.at[ebuf.at[b, 1]], ssem[b], add=True)

        bw = (b + 2) % NBUF

        @pl.when(k >= 2)
        def _():
            pltpu.make_async_copy(rows[bw], acc.at[ebuf.at[bw, 1]],
                                  ssem[bw]).wait()

        @pl.when(k + 2 < NCHUNK)
        def _():
            pltpu.async_copy(edata.at[erow + k + 2], ebuf.at[bw], isem[bw])

        bn = (b + 1) % NBUF

        @pl.when(k + 1 < NCHUNK)
        def _():
            pltpu.make_async_copy(edata.at[erow], ebuf.at[bn],
                                  isem[bn]).wait()
            pltpu.async_copy(x.at[ebuf.at[bn, 0]], rows[bn], gsem[bn])

    def _quad(q, _):
        for b in range(NBUF):
            _body(q * NBUF + b, b)
        return _

    lax.fori_loop(0, NCHUNK // NBUF, _quad, None)
    for k in range(NCHUNK - NCHUNK % NBUF, NCHUNK):
        _body(k, k % NBUF)
    for k in (NCHUNK - 2, NCHUNK - 1):
        b = k % NBUF
        pltpu.make_async_copy(rows[b], acc.at[ebuf.at[b, 1]], ssem[b]).wait()
    plsc.subcore_barrier()

    def _wc(i, _):
        pltpu.sync_copy(acc.at[pl.ds(s * OWN + i * 80, 80)],
                        out.at[pl.ds(c * N + s * OWN + i * 80, 80)])
        return _

    lax.fori_loop(0, ncp, _wc, None)


def _prop(x, edata, D):
    f = pl.kernel(
        functools.partial(_prop_body, D),
        out_type=jax.ShapeDtypeStruct((2 * N, D), jnp.float32),
        mesh=_mesh(),
        compiler_params=pltpu.CompilerParams(use_tc_tiling_on_sc=False,
                                             needs_layout_passes=False),
        scratch_types=(
            [pltpu.VMEM_SHARED((N, D), jnp.float32),
             pltpu.VMEM((NBUF, 3, CK), jnp.int32)]
            + [pltpu.VMEM((CK, D), jnp.float32)] * NBUF
            + [pltpu.SemaphoreType.DMA] * (3 * NBUF)
        ),
    )
    return f(x, edata)


RB = 1000


def _mix_body(g0, g1, dd, g2, g3, sd, w1, o):
    x = jnp.dot(g0[...], w1[0:16, :], preferred_element_type=jnp.float32)
    x += jnp.dot(g1[...], w1[16:32, :], preferred_element_type=jnp.float32)
    x += jnp.dot(dd[...], w1[32:64, :], preferred_element_type=jnp.float32)
    x += jnp.dot(g2[...], w1[64:80, :], preferred_element_type=jnp.float32)
    x += jnp.dot(g3[...], w1[80:96, :], preferred_element_type=jnp.float32)
    x += jnp.dot(sd[...], w1[96:128, :], preferred_element_type=jnp.float32)
    o[...] = x


def _mix(g0, g1, dd, g2, g3, sd, w1):
    grid = (N // RB,)
    bs_e = pl.BlockSpec((RB, ENT), lambda i: (i, 0))
    bs_d = pl.BlockSpec((RB, 32), lambda i: (i, 0))
    bs_w = pl.BlockSpec((128, 128), lambda i: (0, 0))
    return pl.pallas_call(
        _mix_body,
        grid=grid,
        in_specs=[bs_e, bs_e, bs_d, bs_e, bs_e, bs_d, bs_w],
        out_specs=pl.BlockSpec((RB, 128), lambda i: (i, 0)),
        out_shape=jax.ShapeDtypeStruct((N, 128), jnp.float32),
    )(g0, g1, dd, g2, g3, sd, w1)


def _act_body(pa, pb, b1, w2, o):
    h = jnp.maximum(pa[...] + pb[...] + b1[...], 0.0)
    o[...] = jnp.dot(h, w2[...], preferred_element_type=jnp.float32)


def _act(p1, b1, w2):
    grid = (N // RB,)
    return pl.pallas_call(
        _act_body,
        grid=grid,
        in_specs=[
            pl.BlockSpec((RB, 128), lambda i: (i, 0)),
            pl.BlockSpec((RB, 128), lambda i: (i + N // RB, 0)),
            pl.BlockSpec((1, 128), lambda i: (0, 0)),
            pl.BlockSpec((128, 64), lambda i: (0, 0)),
        ],
        out_specs=pl.BlockSpec((RB, 64), lambda i: (i, 0)),
        out_shape=jax.ShapeDtypeStruct((N, 64), jnp.float32),
    )(p1, p1, b1.reshape(1, 128), w2)


def _fin_body(pa, pb, b2, o):
    o[...] = pa[...] + pb[...] + b2[...]


def _fin(p2, b2):
    grid = (N // RB,)
    return pl.pallas_call(
        _fin_body,
        grid=grid,
        in_specs=[
            pl.BlockSpec((RB, 64), lambda i: (i, 0)),
            pl.BlockSpec((RB, 64), lambda i: (i + N // RB, 0)),
            pl.BlockSpec((1, 64), lambda i: (0, 0)),
        ],
        out_specs=pl.BlockSpec((RB, 64), lambda i: (i, 0)),
        out_shape=jax.ShapeDtypeStruct((N, 64), jnp.float32),
    )(p2, p2, b2.reshape(1, 64))


def kernel(static_dense_x, static_sparse_x, dynamic_dense_x, dynamic_sparse_x,
           edges, weights, static_emb_0, static_emb_1, dyn_emb_0, dyn_emb_1,
           W1, b1, W2, b2):
    ss0 = static_sparse_x[:, 0].astype(jnp.int32)
    ss1 = static_sparse_x[:, 1].astype(jnp.int32)
    ds0 = dynamic_sparse_x[0, :, 0].astype(jnp.int32)
    ds1 = dynamic_sparse_x[0, :, 1].astype(jnp.int32)
    src2 = edges[0, 0].astype(jnp.int32).reshape(NW * NCHUNK, CK)
    dst2 = edges[0, 1].astype(jnp.int32).reshape(NW * NCHUNK, CK)
    w2 = lax.bitcast_convert_type(
        weights[0].reshape(NW * NCHUNK, CK), jnp.int32)
    edata = jnp.stack([src2, dst2, w2], axis=1)
    ddx = dynamic_dense_x[0]

    g0, g1, g2, g3 = _emb_gather(ds0, ds1, ss0, ss1,
                                 dyn_emb_0, dyn_emb_1,
                                 static_emb_0, static_emb_1)
    x1 = _mix(g0, g1, ddx, g2, g3, static_dense_x, W1)
    p1 = _prop(x1, edata, 128)
    g = _act(p1, b1, W2)
    p2 = _prop(g, edata, 64)
    return _fin(p2, b2)


# speedup vs baseline: 10.6920x; 1.6219x over previous
import functools

import jax
import jax.numpy as jnp
from jax import lax
from jax.experimental import pallas as pl
from jax.experimental.pallas import tpu as pltpu
from jax.experimental.pallas import tpu_sc as plsc

N = 10000
E = 320000
VOCAB = 1000
ENT = 16

NC = 2
NS = 16
NW = NC * NS
L = 16

EW = E // NW
CK = 80
NCHUNK = EW // CK
NBUF = 4
NEB = 8

OWN = 640

NB = 80
NBLK = N // NB


def _mesh():
    return plsc.VectorSubcoreMesh(core_axis_name="c", subcore_axis_name="s",
                                  num_cores=NC, num_subcores=NS)


def _emb_body(i0, i1, i2, i3, t0, t1, t2, t3, o0, o1, o2, o3,
              ibuf, gbuf, sem):
    c = lax.axis_index("c")
    s = lax.axis_index("s")
    wid = c * NS + s
    idxs = (i0, i1, i2, i3)
    tabs = (t0, t1, t2, t3)
    outs = (o0, o1, o2, o3)
    for j in range(4):
        blk = wid + j * NW

        @pl.when(blk < NBLK)
        def _():
            r0 = blk * NB
            for t in range(4):
                pltpu.sync_copy(idxs[t].at[pl.ds(r0, NB)], ibuf)
                pltpu.async_copy(tabs[t].at[ibuf], gbuf, sem).wait()
                pltpu.sync_copy(gbuf, outs[t].at[pl.ds(r0, NB)])


def _emb_gather(ds0, ds1, ss0, ss1, dt0, dt1, st0, st1):
    f = pl.kernel(
        _emb_body,
        out_type=tuple(jax.ShapeDtypeStruct((N, ENT), jnp.float32)
                       for _ in range(4)),
        mesh=_mesh(),
        compiler_params=pltpu.CompilerParams(use_tc_tiling_on_sc=False),
        scratch_types=[
            pltpu.VMEM((NB,), jnp.int32),
            pltpu.VMEM((NB, ENT), jnp.float32),
            pltpu.SemaphoreType.DMA,
        ],
    )
    return f(ds0, ds1, ss0, ss1, dt0, dt1, st0, st1)


def _prop_body(D, x, edata, out, acc, ebuf,
               r0, r1, r2, r3, g0, g1, g2, g3, s0, s1, s2, s3,
               i0, i1, i2, i3, i4, i5, i6, i7):
    c = lax.axis_index("c")
    s = lax.axis_index("s")
    wid = c * NS + s
    NV = D // L
    rows = (r0, r1, r2, r3)
    gsem = (g0, g1, g2, g3)
    ssem = (s0, s1, s2, s3)
    isem = (i0, i1, i2, i3, i4, i5, i6, i7)

    def _z(i, _):
        for v in range(NV):
            r0[i, pl.ds(v * L, L)] = jnp.zeros((L,), jnp.float32)
        return _

    lax.fori_loop(0, CK, _z, None)
    ncp = jnp.where(s < NS - 1, OWN // 80, (N - (NS - 1) * OWN) // 80)

    def _zc(i, _):
        pltpu.sync_copy(r0, acc.at[pl.ds(s * OWN + i * 80, 80)])
        return _

    lax.fori_loop(0, ncp, _zc, None)

    erow = wid * NCHUNK
    pltpu.sync_copy(edata.at[erow], ebuf.at[0])
    for j in range(1, 4):
        pltpu.async_copy(edata.at[erow + j], ebuf.at[j], isem[j])
    pltpu.async_copy(x.at[ebuf.at[0, 0]], r0, g0)
    pltpu.make_async_copy(edata.at[erow], ebuf.at[1], isem[1]).wait()
    pltpu.async_copy(x.at[ebuf.at[1, 0]], r1, g1)
    plsc.subcore_barrier()

    def _body(k, b, eb):
        bw = (b + 2) % NBUF
        ew = (eb + 2) % NEB

        @pl.when(k >= 2)
        def _():
            pltpu.make_async_copy(rows[bw], acc.at[ebuf.at[ew, 1]],
                                  ssem[bw]).wait()

        @pl.when(k + 4 < NCHUNK)
        def _():
            pltpu.async_copy(edata.at[erow + k + 4], ebuf.at[(eb + 4) % NEB],
                             isem[(eb + 4) % NEB])

        @pl.when(k + 2 < NCHUNK)
        def _():
            pltpu.make_async_copy(edata.at[erow], ebuf.at[ew],
                                  isem[ew]).wait()
            pltpu.async_copy(x.at[ebuf.at[ew, 0]], rows[bw], gsem[bw])

        pltpu.make_async_copy(x.at[ebuf.at[eb, 0]], rows[b], gsem[b]).wait()

        def _scale(e, _):
            wspl = plsc.bitcast(
                plsc.load_gather(
                    ebuf, [jnp.full((L,), eb, jnp.int32),
                           jnp.full((L,), 2, jnp.int32),
                           jnp.full((L,), e, jnp.int32)]), jnp.float32)
            for v in range(NV):
                sl = pl.ds(v * L, L)
                rows[b][e, sl] = rows[b][e, sl] * wspl
            return _

        lax.fori_loop(0, CK, _scale, None, unroll=5)
        pltpu.async_copy(rows[b], acc.at[ebuf.at[eb, 1]], ssem[b], add=True)

    def _oct(q, _):
        for j in range(NEB):
            _body(q * NEB + j, j % NBUF, j)
        return _

    lax.fori_loop(0, NCHUNK // NEB, _oct, None)
    for k in range(NCHUNK - NCHUNK % NEB, NCHUNK):
        _body(k, k % NBUF, k % NEB)
    for k in (NCHUNK - 2, NCHUNK - 1):
        b, eb = k % NBUF, k % NEB
        pltpu.make_async_copy(rows[b], acc.at[ebuf.at[eb, 1]],
                              ssem[b]).wait()
    plsc.subcore_barrier()

    def _wc(i, _):
        pltpu.sync_copy(acc.at[pl.ds(s * OWN + i * 80, 80)],
                        out.at[pl.ds(c * N + s * OWN + i * 80, 80)])
        return _

    lax.fori_loop(0, ncp, _wc, None)


def _prop(x, edata, D):
    f = pl.kernel(
        functools.partial(_prop_body, D),
        out_type=jax.ShapeDtypeStruct((2 * N, D), jnp.float32),
        mesh=_mesh(),
        compiler_params=pltpu.CompilerParams(use_tc_tiling_on_sc=False,
                                             needs_layout_passes=False),
        scratch_types=(
            [pltpu.VMEM_SHARED((N, D), jnp.float32),
             pltpu.VMEM((NEB, 3, CK), jnp.int32)]
            + [pltpu.VMEM((CK, D), jnp.float32)] * NBUF
            + [pltpu.SemaphoreType.DMA] * (2 * NBUF + NEB)
        ),
    )
    return f(x, edata)


RB = 1000


def _mix_body(g0, g1, dd, g2, g3, sd, w1, o):
    x = jnp.dot(g0[...], w1[0:16, :], preferred_element_type=jnp.float32)
    x += jnp.dot(g1[...], w1[16:32, :], preferred_element_type=jnp.float32)
    x += jnp.dot(dd[...], w1[32:64, :], preferred_element_type=jnp.float32)
    x += jnp.dot(g2[...], w1[64:80, :], preferred_element_type=jnp.float32)
    x += jnp.dot(g3[...], w1[80:96, :], preferred_element_type=jnp.float32)
    x += jnp.dot(sd[...], w1[96:128, :], preferred_element_type=jnp.float32)
    o[...] = x


def _mix(g0, g1, dd, g2, g3, sd, w1):
    grid = (N // RB,)
    bs_e = pl.BlockSpec((RB, ENT), lambda i: (i, 0))
    bs_d = pl.BlockSpec((RB, 32), lambda i: (i, 0))
    bs_w = pl.BlockSpec((128, 128), lambda i: (0, 0))
    return pl.pallas_call(
        _mix_body,
        grid=grid,
        in_specs=[bs_e, bs_e, bs_d, bs_e, bs_e, bs_d, bs_w],
        out_specs=pl.BlockSpec((RB, 128), lambda i: (i, 0)),
        out_shape=jax.ShapeDtypeStruct((N, 128), jnp.float32),
    )(g0, g1, dd, g2, g3, sd, w1)


def _act_body(pa, pb, b1, w2, o):
    h = jnp.maximum(pa[...] + pb[...] + b1[...], 0.0)
    o[...] = jnp.dot(h, w2[...], preferred_element_type=jnp.float32)


def _act(p1, b1, w2):
    grid = (N // RB,)
    return pl.pallas_call(
        _act_body,
        grid=grid,
        in_specs=[
            pl.BlockSpec((RB, 128), lambda i: (i, 0)),
            pl.BlockSpec((RB, 128), lambda i: (i + N // RB, 0)),
            pl.BlockSpec((1, 128), lambda i: (0, 0)),
            pl.BlockSpec((128, 64), lambda i: (0, 0)),
        ],
        out_specs=pl.BlockSpec((RB, 64), lambda i: (i, 0)),
        out_shape=jax.ShapeDtypeStruct((N, 64), jnp.float32),
    )(p1, p1, b1.reshape(1, 128), w2)


def _fin_body(pa, pb, b2, o):
    o[...] = pa[...] + pb[...] + b2[...]


def _fin(p2, b2):
    grid = (N // RB,)
    return pl.pallas_call(
        _fin_body,
        grid=grid,
        in_specs=[
            pl.BlockSpec((RB, 64), lambda i: (i, 0)),
            pl.BlockSpec((RB, 64), lambda i: (i + N // RB, 0)),
            pl.BlockSpec((1, 64), lambda i: (0, 0)),
        ],
        out_specs=pl.BlockSpec((RB, 64), lambda i: (i, 0)),
        out_shape=jax.ShapeDtypeStruct((N, 64), jnp.float32),
    )(p2, p2, b2.reshape(1, 64))


def kernel(static_dense_x, static_sparse_x, dynamic_dense_x, dynamic_sparse_x,
           edges, weights, static_emb_0, static_emb_1, dyn_emb_0, dyn_emb_1,
           W1, b1, W2, b2):
    ss0 = static_sparse_x[:, 0].astype(jnp.int32)
    ss1 = static_sparse_x[:, 1].astype(jnp.int32)
    ds0 = dynamic_sparse_x[0, :, 0].astype(jnp.int32)
    ds1 = dynamic_sparse_x[0, :, 1].astype(jnp.int32)
    src2 = edges[0, 0].astype(jnp.int32).reshape(NW * NCHUNK, CK)
    dst2 = edges[0, 1].astype(jnp.int32).reshape(NW * NCHUNK, CK)
    w2 = lax.bitcast_convert_type(
        weights[0].reshape(NW * NCHUNK, CK), jnp.int32)
    edata = jnp.stack([src2, dst2, w2], axis=1)
    ddx = dynamic_dense_x[0]

    g0, g1, g2, g3 = _emb_gather(ds0, ds1, ss0, ss1,
                                 dyn_emb_0, dyn_emb_1,
                                 static_emb_0, static_emb_1)
    x1 = _mix(g0, g1, ddx, g2, g3, static_dense_x, W1)
    p1 = _prop(x1, edata, 128)
    g = _act(p1, b1, W2)
    p2 = _prop(g, edata, 64)
    return _fin(p2, b2)
